# baseline (speedup 1.0000x reference)
import functools

import jax
import jax.numpy as jnp
from jax import lax
from jax.experimental import pallas as pl
from jax.experimental.pallas import tpu as pltpu
from jax.experimental.pallas import tpu_sc as plsc

INNER = 256
NC = 19
QL = 2975
QLP = 2976
HALF = QLP // 2
TEMP = 0.2

NW = 32
ROWS_W = INNER // NW
CHUNKS = QLP // 32
IN_WORDS = ROWS_W * QL
OUT_WORDS = ROWS_W * HALF


def _sc_pack_body(q_hbm, out_hbm, in0, in1, ob0, ob1, sin0, sin1, so0, so1):
    wid = lax.axis_index("s") * 2 + lax.axis_index("c")
    base_rows = wid * ROWS_W

    inbufs = (in0, in1)
    obufs = (ob0, ob1)
    sins = (sin0, sin1)
    souts = (so0, so1)

    def in_off(c):
        return (c * INNER + base_rows) * QL

    def out_off(c):
        return (c * INNER + base_rows) * HALF

    def start_in(c, buf, sem):
        pltpu.make_async_copy(
            q_hbm.at[pl.ds(in_off(c), IN_WORDS)], buf.at[pl.ds(0, IN_WORDS)],
            sem).start()

    def pack_tile(buf, obuf):
        def rne(u):
            return u + jnp.uint32(0x7FFF) + ((u >> jnp.uint32(16)) & jnp.uint32(1))

        def row_body(r, _):
            def chunk_body(k, _):
                o = r * QL + k * 32
                v0 = buf[pl.ds(o, 16)]
                v1 = buf[pl.ds(o + 16, 16)]
                u0 = rne(plsc.bitcast(v0, jnp.uint32))
                u1 = rne(plsc.bitcast(v1, jnp.uint32))
                w = (u0 >> jnp.uint32(16)) | (u1 & jnp.uint32(0xFFFF0000))
                obuf[pl.ds(r * HALF + k * 16, 16)] = w
                return 0
            return lax.fori_loop(0, CHUNKS, chunk_body, 0)
        lax.fori_loop(0, ROWS_W, row_body, 0)

    start_in(0, inbufs[0], sins[0])
    for c in range(NC):
        p = c % 2
        if c + 1 < NC:
            start_in(c + 1, inbufs[1 - p], sins[1 - p])
        pltpu.make_async_copy(
            q_hbm.at[pl.ds(in_off(c), IN_WORDS)],
            inbufs[p].at[pl.ds(0, IN_WORDS)], sins[p]).wait()
        if c >= 2:
            pltpu.make_async_copy(
                obufs[p].at[pl.ds(0, OUT_WORDS)],
                out_hbm.at[pl.ds(out_off(c - 2), OUT_WORDS)], souts[p]).wait()
        pack_tile(inbufs[p], obufs[p])
        pltpu.make_async_copy(
            obufs[p].at[pl.ds(0, OUT_WORDS)],
            out_hbm.at[pl.ds(out_off(c), OUT_WORDS)], souts[p]).start()
    for c in (NC - 2, NC - 1):
        pltpu.make_async_copy(
            obufs[c % 2].at[pl.ds(0, OUT_WORDS)],
            out_hbm.at[pl.ds(out_off(c), OUT_WORDS)], souts[c % 2]).wait()


def _sc_pack(queues):
    qflat = queues.reshape(-1)
    k = functools.partial(
        pl.kernel,
        mesh=plsc.VectorSubcoreMesh(core_axis_name="c", subcore_axis_name="s"),
        compiler_params=pltpu.CompilerParams(needs_layout_passes=False),
        out_type=jax.ShapeDtypeStruct((NC * INNER * HALF,), jnp.uint32),
        scratch_types=[
            pltpu.VMEM((IN_WORDS + 16,), jnp.float32),
            pltpu.VMEM((IN_WORDS + 16,), jnp.float32),
            pltpu.VMEM((OUT_WORDS,), jnp.uint32),
            pltpu.VMEM((OUT_WORDS,), jnp.uint32),
            pltpu.SemaphoreType.DMA,
            pltpu.SemaphoreType.DMA,
            pltpu.SemaphoreType.DMA,
            pltpu.SemaphoreType.DMA,
        ],
    )(_sc_pack_body)
    return k(qflat).reshape(NC, INNER, HALF)


def _seg_kernel(res_ref, fea_ref, sums_ref, cnts_ref):
    b = pl.program_id(0)
    h = pl.program_id(1)
    res = res_ref[0]
    fea = fea_ref[0]
    hw = res.shape[1]
    maxv = jnp.max(res, axis=0, keepdims=True)
    iota = jax.lax.broadcasted_iota(jnp.int32, (NC, hw), 0)
    idx = jnp.where(res == maxv, iota, NC)
    pred = jnp.min(idx, axis=0, keepdims=True)
    onehot = (iota == pred).astype(jnp.float32)
    part = jax.lax.dot_general(
        onehot, fea, (((1,), (1,)), ((), ())),
        preferred_element_type=jnp.float32,
        precision=jax.lax.Precision.DEFAULT)
    pcnt = jnp.sum(onehot, axis=1, keepdims=True)

    @pl.when((b == 0) & (h == 0))
    def _():
        sums_ref[...] = jnp.zeros_like(sums_ref)
        cnts_ref[...] = jnp.zeros_like(cnts_ref)

    sums_ref[...] += part
    cnts_ref[...] += pcnt


def _key_vec(sums_row, cnt):
    key = sums_row / jnp.maximum(cnt, 1.0)
    nrm = jnp.sqrt(jnp.sum(key * key, axis=1, keepdims=True))
    key = key / jnp.maximum(nrm, 1e-12)
    r = jax.lax.broadcasted_iota(jnp.int32, (INNER, INNER), 0)
    col = jax.lax.broadcasted_iota(jnp.int32, (INNER, INNER), 1)
    eye = (r == col).astype(jnp.float32)
    keyT = jax.lax.dot_general(
        eye, key, (((1,), (1,)), ((), ())),
        preferred_element_type=jnp.float32,
        precision=jax.lax.Precision.HIGHEST)
    return keyT * (1.0 / TEMP)


def _unpack_halves(u):
    qa = pltpu.bitcast(u << jnp.uint32(16), jnp.float32)
    qb = pltpu.bitcast(u & jnp.uint32(0xFFFF0000), jnp.float32)
    return qa, qb


def _loss_kernel(sums_ref, cnts_ref, q_ref, loss_ref,
                 stot_a_ref, stot_b_ref, qcache_ref,
                 mpos_ref, spos_ref, x0_ref):
    ph = pl.program_id(0)
    c = pl.program_id(1)

    @pl.when((ph == 0) & (c == 0))
    def _():
        stot_a_ref[...] = jnp.zeros_like(stot_a_ref)
        stot_b_ref[...] = jnp.zeros_like(stot_b_ref)
        loss_ref[...] = jnp.zeros_like(loss_ref)

    pad = jax.lax.broadcasted_iota(jnp.int32, (INNER, HALF), 1) == (HALF - 1)

    @pl.when(ph == 0)
    def _():
        u = q_ref[0]
        qa, qb = _unpack_halves(u)
        stot_a_ref[...] += qa
        stot_b_ref[...] += qb
        qcache_ref[c] = u
        a = _key_vec(sums_ref[0], cnts_ref[0])
        xa = a * qa
        xb = jnp.where(pad, -1e30, a * qb)
        mp = jnp.maximum(jnp.max(xa, axis=1, keepdims=True),
                         jnp.max(xb, axis=1, keepdims=True))
        sp = (jnp.sum(jnp.exp(xa - mp), axis=1, keepdims=True) +
              jnp.sum(jnp.exp(xb - mp), axis=1, keepdims=True))
        mpos_ref[c] = mp
        spos_ref[c] = sp
        x0_ref[c] = xa[:, 0:1]

    @pl.when(ph == 1)
    def _():
        u = qcache_ref[c]
        qa, qb = _unpack_halves(u)
        a = _key_vec(sums_ref[0], cnts_ref[0])
        xna = a * (stot_a_ref[...] - qa)
        xnb = jnp.where(pad, -1e30, a * (stot_b_ref[...] - qb))
        mn = jnp.maximum(jnp.max(xna, axis=1, keepdims=True),
                         jnp.max(xnb, axis=1, keepdims=True))
        sn = (jnp.sum(jnp.exp(xna - mn), axis=1, keepdims=True) +
              jnp.sum(jnp.exp(xnb - mn), axis=1, keepdims=True))
        mp = mpos_ref[c]
        sp = spos_ref[c]
        m = jnp.maximum(mp, mn)
        s = sp * jnp.exp(mp - m) + sn * jnp.exp(mn - m)
        lse = m + jnp.log(s)
        loss_c = jnp.sum(lse - x0_ref[c], axis=0, keepdims=True) / INNER
        cnt = cnts_ref[0]
        loss_ref[...] += jnp.where(cnt > 0.0, loss_c[0:1, :], 0.0)


def kernel(fea, res, queues):
    bs = fea.shape[0]
    hw = fea.shape[2] * fea.shape[3]
    hw2 = hw // 2
    fea_r = fea.reshape(bs, INNER, hw)
    res_r = res.reshape(bs, NC, hw)

    sums, cnts = pl.pallas_call(
        _seg_kernel,
        grid=(bs, 2),
        in_specs=[
            pl.BlockSpec((1, NC, hw2), lambda b, h: (b, 0, h)),
            pl.BlockSpec((1, INNER, hw2), lambda b, h: (b, 0, h)),
        ],
        out_specs=[
            pl.BlockSpec((NC, INNER), lambda b, h: (0, 0)),
            pl.BlockSpec((NC, 1), lambda b, h: (0, 0)),
        ],
        out_shape=[
            jax.ShapeDtypeStruct((NC, INNER), jnp.float32),
            jax.ShapeDtypeStruct((NC, 1), jnp.float32),
        ],
    )(res_r, fea_r)

    sums3 = sums.reshape(NC, 1, INNER)
    cnts3 = cnts.reshape(NC, 1, 1)
    qpk = _sc_pack(queues)

    loss = pl.pallas_call(
        _loss_kernel,
        grid=(2, NC),
        in_specs=[
            pl.BlockSpec((1, 1, INNER), lambda ph, c: (c, 0, 0)),
            pl.BlockSpec((1, 1, 1), lambda ph, c: (c, 0, 0)),
            pl.BlockSpec((1, INNER, HALF), lambda ph, c: (c * (1 - ph), 0, 0)),
        ],
        out_specs=pl.BlockSpec((1, 1), lambda ph, c: (0, 0)),
        out_shape=jax.ShapeDtypeStruct((1, 1), jnp.float32),
        scratch_shapes=[
            pltpu.VMEM((INNER, HALF), jnp.float32),
            pltpu.VMEM((INNER, HALF), jnp.float32),
            pltpu.VMEM((NC, INNER, HALF), jnp.uint32),
            pltpu.VMEM((NC, INNER, 1), jnp.float32),
            pltpu.VMEM((NC, INNER, 1), jnp.float32),
            pltpu.VMEM((NC, INNER, 1), jnp.float32),
        ],
    )(sums3, cnts3, qpk)

    return (res, loss[0, 0])


# speedup vs baseline: 1.7316x; 1.7316x over previous
import jax
import jax.numpy as jnp
from jax.experimental import pallas as pl
from jax.experimental.pallas import tpu as pltpu

INNER = 256
NC = 19
QL = 2975
TEMP = 0.2


def _seg_kernel(res_ref, fea_ref, sums_ref, cnts_ref):
    b = pl.program_id(0)
    h = pl.program_id(1)
    res = res_ref[0]
    fea = fea_ref[0]
    hw = res.shape[1]
    maxv = jnp.max(res, axis=0, keepdims=True)
    iota = jax.lax.broadcasted_iota(jnp.int32, (NC, hw), 0)
    idx = jnp.where(res == maxv, iota, NC)
    pred = jnp.min(idx, axis=0, keepdims=True)
    onehot = (iota == pred).astype(jnp.float32)
    part = jax.lax.dot_general(
        onehot, fea, (((1,), (1,)), ((), ())),
        preferred_element_type=jnp.float32,
        precision=jax.lax.Precision.DEFAULT)
    pcnt = jnp.sum(onehot, axis=1, keepdims=True)

    @pl.when((b == 0) & (h == 0))
    def _():
        sums_ref[...] = jnp.zeros_like(sums_ref)
        cnts_ref[...] = jnp.zeros_like(cnts_ref)

    sums_ref[...] += part
    cnts_ref[...] += pcnt


def _key_vec(sums_row, cnt):
    key = sums_row / jnp.maximum(cnt, 1.0)
    nrm = jnp.sqrt(jnp.sum(key * key, axis=1, keepdims=True))
    key = key / jnp.maximum(nrm, 1e-12)
    r = jax.lax.broadcasted_iota(jnp.int32, (INNER, INNER), 0)
    col = jax.lax.broadcasted_iota(jnp.int32, (INNER, INNER), 1)
    eye = (r == col).astype(jnp.float32)
    keyT = jax.lax.dot_general(
        eye, key, (((1,), (1,)), ((), ())),
        preferred_element_type=jnp.float32,
        precision=jax.lax.Precision.HIGHEST)
    return keyT * (1.0 / TEMP)


def _loss_kernel(sums_ref, cnts_ref, q_ref, loss_ref,
                 stot_ref, qcache_ref, mpos_ref, spos_ref, x0_ref):
    ph = pl.program_id(0)
    c = pl.program_id(1)

    @pl.when((ph == 0) & (c == 0))
    def _():
        stot_ref[...] = jnp.zeros_like(stot_ref)
        loss_ref[...] = jnp.zeros_like(loss_ref)

    @pl.when(ph == 0)
    def _():
        q0 = q_ref[0]
        stot_ref[...] += q0
        qcache_ref[c] = q0.astype(jnp.bfloat16)
        a = _key_vec(sums_ref[0], cnts_ref[0])
        xp = a * q0
        mp = jnp.max(xp, axis=1, keepdims=True)
        sp = jnp.sum(jnp.exp(xp - mp), axis=1, keepdims=True)
        mpos_ref[c] = mp
        spos_ref[c] = sp
        x0_ref[c] = xp[:, 0:1]

    @pl.when(ph == 1)
    def _():
        q = qcache_ref[c].astype(jnp.float32)
        a = _key_vec(sums_ref[0], cnts_ref[0])
        xn = a * (stot_ref[...] - q)
        mn = jnp.max(xn, axis=1, keepdims=True)
        sn = jnp.sum(jnp.exp(xn - mn), axis=1, keepdims=True)
        mp = mpos_ref[c]
        sp = spos_ref[c]
        m = jnp.maximum(mp, mn)
        s = sp * jnp.exp(mp - m) + sn * jnp.exp(mn - m)
        lse = m + jnp.log(s)
        loss_c = jnp.sum(lse - x0_ref[c], axis=0, keepdims=True) / INNER
        cnt = cnts_ref[0]
        loss_ref[...] += jnp.where(cnt > 0.0, loss_c[0:1, :], 0.0)


def kernel(fea, res, queues):
    bs = fea.shape[0]
    hw = fea.shape[2] * fea.shape[3]
    hw2 = hw // 2
    fea_r = fea.reshape(bs, INNER, hw)
    res_r = res.reshape(bs, NC, hw)

    sums, cnts = pl.pallas_call(
        _seg_kernel,
        grid=(bs, 2),
        in_specs=[
            pl.BlockSpec((1, NC, hw2), lambda b, h: (b, 0, h)),
            pl.BlockSpec((1, INNER, hw2), lambda b, h: (b, 0, h)),
        ],
        out_specs=[
            pl.BlockSpec((NC, INNER), lambda b, h: (0, 0)),
            pl.BlockSpec((NC, 1), lambda b, h: (0, 0)),
        ],
        out_shape=[
            jax.ShapeDtypeStruct((NC, INNER), jnp.float32),
            jax.ShapeDtypeStruct((NC, 1), jnp.float32),
        ],
    )(res_r, fea_r)

    sums3 = sums.reshape(NC, 1, INNER)
    cnts3 = cnts.reshape(NC, 1, 1)

    loss = pl.pallas_call(
        _loss_kernel,
        grid=(2, NC),
        in_specs=[
            pl.BlockSpec((1, 1, INNER), lambda ph, c: (c, 0, 0)),
            pl.BlockSpec((1, 1, 1), lambda ph, c: (c, 0, 0)),
            pl.BlockSpec((1, INNER, QL), lambda ph, c: (c * (1 - ph), 0, 0)),
        ],
        out_specs=pl.BlockSpec((1, 1), lambda ph, c: (0, 0)),
        out_shape=jax.ShapeDtypeStruct((1, 1), jnp.float32),
        scratch_shapes=[
            pltpu.VMEM((INNER, QL), jnp.float32),
            pltpu.VMEM((NC, INNER, QL), jnp.bfloat16),
            pltpu.VMEM((NC, INNER, 1), jnp.float32),
            pltpu.VMEM((NC, INNER, 1), jnp.float32),
            pltpu.VMEM((NC, INNER, 1), jnp.float32),
        ],
    )(sums3, cnts3, queues)

    return (res, loss[0, 0])
